# baseline (device time: 33243 ns/iter reference)
import jax
import jax.numpy as jnp
from jax import lax
from jax.experimental import pallas as pl
from jax.experimental.pallas import tpu as pltpu

N_DEV = 4
G = 8
M = 1024
MPS = M + 8 * G
GBITS = [1 << b for b in range(7, -1, -1)]
NB = len(GBITS)


def _a2av_fused(x, rank_row, ot, cnt_row):
    m, n = x.shape

    def body(x_ref, rank_ref, ot_ref, cin_ref, out_ref,
             xsp_ref, bnd_ref, cbuf_ref, csm_ref,
             csend, crecv, send_sems, recv_sems, bsend, brecv,
             local_sems, lbnd, csm_sem):
        my_x = lax.axis_index("x")
        my_y = lax.axis_index("y")
        me = lax.axis_index("z")

        barrier_sem = pltpu.get_barrier_semaphore()
        for delta in range(1, N_DEV):
            pl.semaphore_signal(
                barrier_sem, inc=1,
                device_id=(my_x, my_y, (me + delta) % N_DEV),
                device_id_type=pl.DeviceIdType.MESH,
            )
        pl.semaphore_wait(barrier_sem, N_DEV - 1)

        cbuf_ref[pl.ds(me, 1)] = cin_ref[...]
        for delta in range(1, N_DEV):
            d = (me + delta) % N_DEV
            rdma = pltpu.make_async_remote_copy(
                src_ref=cbuf_ref.at[pl.ds(me, 1)],
                dst_ref=cbuf_ref.at[pl.ds(me, 1)],
                send_sem=csend.at[delta - 1],
                recv_sem=crecv.at[me],
                device_id=(my_x, my_y, d),
                device_id_type=pl.DeviceIdType.MESH,
            )
            rdma.start()
        for delta in range(1, N_DEV):
            s = (me - delta) % N_DEV
            desc = pltpu.make_async_remote_copy(
                src_ref=cbuf_ref.at[pl.ds(0, 1)],
                dst_ref=cbuf_ref.at[pl.ds(s, 1)],
                send_sem=csend.at[delta - 1],
                recv_sem=crecv.at[s],
                device_id=(my_x, my_y, s),
                device_id_type=pl.DeviceIdType.MESH,
            )
            desc.wait_recv()
        cp = pltpu.make_async_copy(cbuf_ref, csm_ref, csm_sem)
        cp.start()
        cp.wait()
        for delta in range(1, N_DEV):
            d = (me + delta) % N_DEV
            desc = pltpu.make_async_remote_copy(
                src_ref=cbuf_ref.at[pl.ds(me, 1)],
                dst_ref=cbuf_ref.at[pl.ds(me, 1)],
                send_sem=csend.at[delta - 1],
                recv_sem=crecv.at[me],
                device_id=(my_x, my_y, d),
                device_id_type=pl.DeviceIdType.MESH,
            )
            desc.wait_send()

        def C(src, dst):
            return csm_ref[src, 0, dst]

        def bstart(src, dst):
            acc = jnp.int32(0)
            for z in range(N_DEV):
                acc += jnp.where(z < src, C(z, dst), 0)
            return acc

        def rows8(goff):
            return pl.multiple_of(goff * G, G)

        phi = []
        cme = []
        off = []
        o_acc = jnp.int32(0)
        for d in range(N_DEV):
            b = bstart(me, d)
            c = C(me, d)
            phi.append(b % G)
            cme.append(c)
            off.append(o_acc)
            o_acc += (phi[d] + c + (G - 1)) // G

        def sel(lst, idx):
            acc = jnp.int32(0)
            for k_ in range(N_DEV):
                acc = jnp.where(idx == k_, lst[k_], acc)
            return acc

        tgt_row = rank_ref[...]
        for d in range(N_DEV):
            tgt_row += ot_ref[pl.ds(d, 1), :] * (off[d] * G + phi[d])
        q_iota = lax.broadcasted_iota(jnp.int32, (MPS, m), 0)
        pm = (q_iota == tgt_row).astype(jnp.bfloat16)
        xsp_ref[...] = jnp.dot(
            pm, x_ref[...].astype(jnp.bfloat16),
            preferred_element_type=jnp.float32,
        )

        def interior(src, dst):
            b = bstart(src, dst)
            c = C(src, dst)
            f_g = (b + (G - 1)) // G
            l_g = (b + c) // G
            return f_g, jnp.maximum(l_g - f_g, 0)

        for delta in range(1, N_DEV):
            d = (me + delta) % N_DEV
            b = bstart(me, d)
            f_g, gint = interior(me, d)
            off_d = sel(off, d)
            sg0 = off_d + f_g - b // G
            part = jnp.int32(0)
            for kb, k in enumerate(GBITS):
                bit_on = (gint & k) != 0

                @pl.when(bit_on)
                def _(k=k, kb=kb, delta=delta, d=d, sg0=sg0, f_g=f_g,
                      part=part):
                    rdma = pltpu.make_async_remote_copy(
                        src_ref=xsp_ref.at[pl.ds(rows8(sg0 + part), k * G), :],
                        dst_ref=out_ref.at[pl.ds(rows8(f_g + part), k * G), :],
                        send_sem=send_sems.at[delta - 1, kb],
                        recv_sem=recv_sems.at[me, kb],
                        device_id=(my_x, my_y, d),
                        device_id_type=pl.DeviceIdType.MESH,
                    )
                    rdma.start()

                part += jnp.where(bit_on, jnp.int32(k), 0)

            tg = off_d + (sel(phi, d) + jnp.maximum(sel(cme, d) - 1, 0)) // G
            for j, sg in enumerate((off_d, tg)):
                rdma = pltpu.make_async_remote_copy(
                    src_ref=xsp_ref.at[pl.ds(rows8(sg), G), :],
                    dst_ref=bnd_ref.at[pl.ds(rows8(2 * me + j), G), :],
                    send_sem=bsend.at[delta - 1, j],
                    recv_sem=brecv.at[me, j],
                    device_id=(my_x, my_y, d),
                    device_id_type=pl.DeviceIdType.MESH,
                )
                rdma.start()

        b_self = bstart(me, me)
        f_gs, gint_self = interior(me, me)
        off_me = sel(off, me)
        sg0s = off_me + f_gs - b_self // G
        part = jnp.int32(0)
        for kb, k in enumerate(GBITS):
            bit_on = (gint_self & k) != 0

            @pl.when(bit_on)
            def _(k=k, kb=kb, sg0s=sg0s, f_gs=f_gs, part=part):
                cp2 = pltpu.make_async_copy(
                    xsp_ref.at[pl.ds(rows8(sg0s + part), k * G), :],
                    out_ref.at[pl.ds(rows8(f_gs + part), k * G), :],
                    local_sems.at[kb],
                )
                cp2.start()

            part += jnp.where(bit_on, jnp.int32(k), 0)
        tgs = off_me + (sel(phi, me) + jnp.maximum(sel(cme, me) - 1, 0)) // G
        for j, sg in enumerate((off_me, tgs)):
            cp3 = pltpu.make_async_copy(
                xsp_ref.at[pl.ds(rows8(sg), G), :],
                bnd_ref.at[pl.ds(rows8(2 * me + j), G), :],
                lbnd.at[j],
            )
            cp3.start()

        for delta in range(1, N_DEV):
            s = (me - delta) % N_DEV
            f_g, gint = interior(s, me)
            part = jnp.int32(0)
            for kb, k in enumerate(GBITS):
                bit_on = (gint & k) != 0

                @pl.when(bit_on)
                def _(k=k, kb=kb, delta=delta, s=s, f_g=f_g, part=part):
                    desc = pltpu.make_async_remote_copy(
                        src_ref=xsp_ref.at[pl.ds(0, k * G), :],
                        dst_ref=out_ref.at[pl.ds(rows8(f_g + part), k * G), :],
                        send_sem=send_sems.at[delta - 1, kb],
                        recv_sem=recv_sems.at[s, kb],
                        device_id=(my_x, my_y, s),
                        device_id_type=pl.DeviceIdType.MESH,
                    )
                    desc.wait_recv()

                part += jnp.where(bit_on, jnp.int32(k), 0)
            for j in range(2):
                desc = pltpu.make_async_remote_copy(
                    src_ref=xsp_ref.at[pl.ds(0, G), :],
                    dst_ref=bnd_ref.at[pl.ds(rows8(2 * s + j), G), :],
                    send_sem=bsend.at[delta - 1, j],
                    recv_sem=brecv.at[s, j],
                    device_id=(my_x, my_y, s),
                    device_id_type=pl.DeviceIdType.MESH,
                )
                desc.wait_recv()
        for kb, k in enumerate(GBITS):
            bit_on = (gint_self & k) != 0

            @pl.when(bit_on)
            def _(k=k, kb=kb):
                pltpu.make_async_copy(
                    xsp_ref.at[pl.ds(0, k * G), :],
                    out_ref.at[pl.ds(0, k * G), :],
                    local_sems.at[kb],
                ).wait()
        for j in range(2):
            pltpu.make_async_copy(
                xsp_ref.at[pl.ds(0, G), :],
                bnd_ref.at[pl.ds(rows8(j), G), :],
                lbnd.at[j],
            ).wait()

        sub_iota = lax.broadcasted_iota(jnp.int32, (G, n), 0)
        for z in range(1, N_DEV):
            b_z = bstart(z, me)
            phi_z = b_z % G
            head = bnd_ref[pl.ds(rows8(2 * z), G), :]
            tail = bnd_ref[pl.ds(rows8(2 * (z - 1) + 1), G), :]
            merged = jnp.where(sub_iota < phi_z, tail, head)
            out_ref[pl.ds(rows8(b_z // G), G), :] = merged

        for delta in range(1, N_DEV):
            d = (me + delta) % N_DEV
            _, gint = interior(me, d)
            for kb, k in enumerate(GBITS):
                bit_on = (gint & k) != 0

                @pl.when(bit_on)
                def _(k=k, kb=kb, delta=delta, d=d):
                    desc = pltpu.make_async_remote_copy(
                        src_ref=xsp_ref.at[pl.ds(0, k * G), :],
                        dst_ref=out_ref.at[pl.ds(0, k * G), :],
                        send_sem=send_sems.at[delta - 1, kb],
                        recv_sem=recv_sems.at[me, kb],
                        device_id=(my_x, my_y, d),
                        device_id_type=pl.DeviceIdType.MESH,
                    )
                    desc.wait_send()
            for j in range(2):
                desc = pltpu.make_async_remote_copy(
                    src_ref=xsp_ref.at[pl.ds(0, G), :],
                    dst_ref=bnd_ref.at[pl.ds(rows8(j), G), :],
                    send_sem=bsend.at[delta - 1, j],
                    recv_sem=brecv.at[me, j],
                    device_id=(my_x, my_y, d),
                    device_id_type=pl.DeviceIdType.MESH,
                )
                desc.wait_send()

    return pl.pallas_call(
        body,
        out_shape=jax.ShapeDtypeStruct((m, n), jnp.float32),
        in_specs=[
            pl.BlockSpec(memory_space=pltpu.VMEM),
            pl.BlockSpec(memory_space=pltpu.VMEM),
            pl.BlockSpec(memory_space=pltpu.VMEM),
            pl.BlockSpec(memory_space=pltpu.VMEM),
        ],
        out_specs=pl.BlockSpec(memory_space=pltpu.VMEM),
        scratch_shapes=[
            pltpu.VMEM((MPS, n), jnp.float32),
            pltpu.VMEM((2 * N_DEV * G, n), jnp.float32),
            pltpu.VMEM((N_DEV, 1, 128), jnp.int32),
            pltpu.SMEM((N_DEV, 1, 128), jnp.int32),
            pltpu.SemaphoreType.DMA((N_DEV - 1,)),
            pltpu.SemaphoreType.DMA((N_DEV,)),
            pltpu.SemaphoreType.DMA((N_DEV - 1, NB)),
            pltpu.SemaphoreType.DMA((N_DEV, NB)),
            pltpu.SemaphoreType.DMA((N_DEV - 1, 2)),
            pltpu.SemaphoreType.DMA((N_DEV, 2)),
            pltpu.SemaphoreType.DMA((NB,)),
            pltpu.SemaphoreType.DMA((2,)),
            pltpu.SemaphoreType.DMA,
        ],
        compiler_params=pltpu.CompilerParams(collective_id=0),
    )(x, rank_row, ot, cnt_row)


def kernel(x, dest):
    m, n = x.shape
    i32 = jnp.int32

    onehot = (dest[:, None] == jnp.arange(N_DEV)[None, :]).astype(i32)
    cnt = jnp.sum(onehot, axis=0, dtype=i32)
    tril = jnp.tril(jnp.ones((m, m), jnp.bfloat16), -1)
    ranks = jnp.dot(tril, onehot.astype(jnp.bfloat16),
                    preferred_element_type=jnp.float32).astype(i32)
    rank_row = jnp.sum(onehot * ranks, axis=1, dtype=i32).reshape(1, m)
    ot = (jnp.arange(N_DEV, dtype=i32)[:, None] == dest[None, :]).astype(i32)

    cnt_row = jnp.zeros((1, 1, 128), i32).at[0, 0, :N_DEV].set(cnt)
    return _a2av_fused(x, rank_row, ot, cnt_row)


# device time: 32524 ns/iter; 1.0221x vs baseline; 1.0221x over previous
import jax
import jax.numpy as jnp
from jax import lax
from jax.experimental import pallas as pl
from jax.experimental.pallas import tpu as pltpu

N_DEV = 4
G = 8
M = 1024
MPS = M + 8 * G
GBITS = [1 << b for b in range(7, -1, -1)]
NB = len(GBITS)


def _a2av_fused(x, rank_row, ot, cnt_row):
    m, n = x.shape

    def body(x_ref, rank_ref, ot_ref, cin_ref, out_ref,
             xsp_ref, bnd_ref, cbuf_ref, csm_ref,
             csend, crecv, send_sems, recv_sems, bsend, brecv,
             local_sems, lbnd, csm_sem):
        my_x = lax.axis_index("x")
        my_y = lax.axis_index("y")
        me = lax.axis_index("z")

        barrier_sem = pltpu.get_barrier_semaphore()
        for delta in range(1, N_DEV):
            pl.semaphore_signal(
                barrier_sem, inc=1,
                device_id=(my_x, my_y, (me + delta) % N_DEV),
                device_id_type=pl.DeviceIdType.MESH,
            )
        pl.semaphore_wait(barrier_sem, N_DEV - 1)

        cbuf_ref[pl.ds(me, 1)] = cin_ref[...]
        for delta in range(1, N_DEV):
            d = (me + delta) % N_DEV
            rdma = pltpu.make_async_remote_copy(
                src_ref=cbuf_ref.at[pl.ds(me, 1)],
                dst_ref=csm_ref.at[pl.ds(me, 1)],
                send_sem=csend.at[delta - 1],
                recv_sem=crecv.at[me],
                device_id=(my_x, my_y, d),
                device_id_type=pl.DeviceIdType.MESH,
            )
            rdma.start()
        cp = pltpu.make_async_copy(
            cbuf_ref.at[pl.ds(me, 1)], csm_ref.at[pl.ds(me, 1)], csm_sem
        )
        cp.start()
        cp.wait()
        for delta in range(1, N_DEV):
            s = (me - delta) % N_DEV
            desc = pltpu.make_async_remote_copy(
                src_ref=cbuf_ref.at[pl.ds(0, 1)],
                dst_ref=csm_ref.at[pl.ds(s, 1)],
                send_sem=csend.at[delta - 1],
                recv_sem=crecv.at[s],
                device_id=(my_x, my_y, s),
                device_id_type=pl.DeviceIdType.MESH,
            )
            desc.wait_recv()
        for delta in range(1, N_DEV):
            d = (me + delta) % N_DEV
            desc = pltpu.make_async_remote_copy(
                src_ref=cbuf_ref.at[pl.ds(me, 1)],
                dst_ref=csm_ref.at[pl.ds(me, 1)],
                send_sem=csend.at[delta - 1],
                recv_sem=crecv.at[me],
                device_id=(my_x, my_y, d),
                device_id_type=pl.DeviceIdType.MESH,
            )
            desc.wait_send()

        def C(src, dst):
            return csm_ref[src, 0, dst]

        def bstart(src, dst):
            acc = jnp.int32(0)
            for z in range(N_DEV):
                acc += jnp.where(z < src, C(z, dst), 0)
            return acc

        def rows8(goff):
            return pl.multiple_of(goff * G, G)

        phi = []
        cme = []
        off = []
        o_acc = jnp.int32(0)
        for d in range(N_DEV):
            b = bstart(me, d)
            c = C(me, d)
            phi.append(b % G)
            cme.append(c)
            off.append(o_acc)
            o_acc += (phi[d] + c + (G - 1)) // G

        def sel(lst, idx):
            acc = jnp.int32(0)
            for k_ in range(N_DEV):
                acc = jnp.where(idx == k_, lst[k_], acc)
            return acc

        tgt_row = rank_ref[...]
        for d in range(N_DEV):
            tgt_row += ot_ref[pl.ds(d, 1), :] * (off[d] * G + phi[d])
        q_iota = lax.broadcasted_iota(jnp.int32, (MPS, m), 0)
        pm = (q_iota == tgt_row).astype(jnp.bfloat16)
        xsp_ref[...] = jnp.dot(
            pm, x_ref[...].astype(jnp.bfloat16),
            preferred_element_type=jnp.float32,
        )

        def interior(src, dst):
            b = bstart(src, dst)
            c = C(src, dst)
            f_g = (b + (G - 1)) // G
            l_g = (b + c) // G
            return f_g, jnp.maximum(l_g - f_g, 0)

        for delta in range(1, N_DEV):
            d = (me + delta) % N_DEV
            b = bstart(me, d)
            f_g, gint = interior(me, d)
            off_d = sel(off, d)
            sg0 = off_d + f_g - b // G
            part = jnp.int32(0)
            for kb, k in enumerate(GBITS):
                bit_on = (gint & k) != 0

                @pl.when(bit_on)
                def _(k=k, kb=kb, delta=delta, d=d, sg0=sg0, f_g=f_g,
                      part=part):
                    rdma = pltpu.make_async_remote_copy(
                        src_ref=xsp_ref.at[pl.ds(rows8(sg0 + part), k * G), :],
                        dst_ref=out_ref.at[pl.ds(rows8(f_g + part), k * G), :],
                        send_sem=send_sems.at[delta - 1, kb],
                        recv_sem=recv_sems.at[me, kb],
                        device_id=(my_x, my_y, d),
                        device_id_type=pl.DeviceIdType.MESH,
                    )
                    rdma.start()

                part += jnp.where(bit_on, jnp.int32(k), 0)

            tg = off_d + (sel(phi, d) + jnp.maximum(sel(cme, d) - 1, 0)) // G
            for j, sg in enumerate((off_d, tg)):
                rdma = pltpu.make_async_remote_copy(
                    src_ref=xsp_ref.at[pl.ds(rows8(sg), G), :],
                    dst_ref=bnd_ref.at[pl.ds(rows8(2 * me + j), G), :],
                    send_sem=bsend.at[delta - 1, j],
                    recv_sem=brecv.at[me, j],
                    device_id=(my_x, my_y, d),
                    device_id_type=pl.DeviceIdType.MESH,
                )
                rdma.start()

        b_self = bstart(me, me)
        f_gs, gint_self = interior(me, me)
        off_me = sel(off, me)
        sg0s = off_me + f_gs - b_self // G
        part = jnp.int32(0)
        for kb, k in enumerate(GBITS):
            bit_on = (gint_self & k) != 0

            @pl.when(bit_on)
            def _(k=k, kb=kb, sg0s=sg0s, f_gs=f_gs, part=part):
                cp2 = pltpu.make_async_copy(
                    xsp_ref.at[pl.ds(rows8(sg0s + part), k * G), :],
                    out_ref.at[pl.ds(rows8(f_gs + part), k * G), :],
                    local_sems.at[kb],
                )
                cp2.start()

            part += jnp.where(bit_on, jnp.int32(k), 0)
        tgs = off_me + (sel(phi, me) + jnp.maximum(sel(cme, me) - 1, 0)) // G
        for j, sg in enumerate((off_me, tgs)):
            cp3 = pltpu.make_async_copy(
                xsp_ref.at[pl.ds(rows8(sg), G), :],
                bnd_ref.at[pl.ds(rows8(2 * me + j), G), :],
                lbnd.at[j],
            )
            cp3.start()

        for delta in range(1, N_DEV):
            s = (me - delta) % N_DEV
            f_g, gint = interior(s, me)
            part = jnp.int32(0)
            for kb, k in enumerate(GBITS):
                bit_on = (gint & k) != 0

                @pl.when(bit_on)
                def _(k=k, kb=kb, delta=delta, s=s, f_g=f_g, part=part):
                    desc = pltpu.make_async_remote_copy(
                        src_ref=xsp_ref.at[pl.ds(0, k * G), :],
                        dst_ref=out_ref.at[pl.ds(rows8(f_g + part), k * G), :],
                        send_sem=send_sems.at[delta - 1, kb],
                        recv_sem=recv_sems.at[s, kb],
                        device_id=(my_x, my_y, s),
                        device_id_type=pl.DeviceIdType.MESH,
                    )
                    desc.wait_recv()

                part += jnp.where(bit_on, jnp.int32(k), 0)
            for j in range(2):
                desc = pltpu.make_async_remote_copy(
                    src_ref=xsp_ref.at[pl.ds(0, G), :],
                    dst_ref=bnd_ref.at[pl.ds(rows8(2 * s + j), G), :],
                    send_sem=bsend.at[delta - 1, j],
                    recv_sem=brecv.at[s, j],
                    device_id=(my_x, my_y, s),
                    device_id_type=pl.DeviceIdType.MESH,
                )
                desc.wait_recv()
        for kb, k in enumerate(GBITS):
            bit_on = (gint_self & k) != 0

            @pl.when(bit_on)
            def _(k=k, kb=kb):
                pltpu.make_async_copy(
                    xsp_ref.at[pl.ds(0, k * G), :],
                    out_ref.at[pl.ds(0, k * G), :],
                    local_sems.at[kb],
                ).wait()
        for j in range(2):
            pltpu.make_async_copy(
                xsp_ref.at[pl.ds(0, G), :],
                bnd_ref.at[pl.ds(rows8(j), G), :],
                lbnd.at[j],
            ).wait()

        sub_iota = lax.broadcasted_iota(jnp.int32, (G, n), 0)
        for z in range(1, N_DEV):
            b_z = bstart(z, me)
            phi_z = b_z % G
            head = bnd_ref[pl.ds(rows8(2 * z), G), :]
            tail = bnd_ref[pl.ds(rows8(2 * (z - 1) + 1), G), :]
            merged = jnp.where(sub_iota < phi_z, tail, head)
            out_ref[pl.ds(rows8(b_z // G), G), :] = merged

        for delta in range(1, N_DEV):
            d = (me + delta) % N_DEV
            _, gint = interior(me, d)
            for kb, k in enumerate(GBITS):
                bit_on = (gint & k) != 0

                @pl.when(bit_on)
                def _(k=k, kb=kb, delta=delta, d=d):
                    desc = pltpu.make_async_remote_copy(
                        src_ref=xsp_ref.at[pl.ds(0, k * G), :],
                        dst_ref=out_ref.at[pl.ds(0, k * G), :],
                        send_sem=send_sems.at[delta - 1, kb],
                        recv_sem=recv_sems.at[me, kb],
                        device_id=(my_x, my_y, d),
                        device_id_type=pl.DeviceIdType.MESH,
                    )
                    desc.wait_send()
            for j in range(2):
                desc = pltpu.make_async_remote_copy(
                    src_ref=xsp_ref.at[pl.ds(0, G), :],
                    dst_ref=bnd_ref.at[pl.ds(rows8(j), G), :],
                    send_sem=bsend.at[delta - 1, j],
                    recv_sem=brecv.at[me, j],
                    device_id=(my_x, my_y, d),
                    device_id_type=pl.DeviceIdType.MESH,
                )
                desc.wait_send()

    return pl.pallas_call(
        body,
        out_shape=jax.ShapeDtypeStruct((m, n), jnp.float32),
        in_specs=[
            pl.BlockSpec(memory_space=pltpu.VMEM),
            pl.BlockSpec(memory_space=pltpu.VMEM),
            pl.BlockSpec(memory_space=pltpu.VMEM),
            pl.BlockSpec(memory_space=pltpu.VMEM),
        ],
        out_specs=pl.BlockSpec(memory_space=pltpu.VMEM),
        scratch_shapes=[
            pltpu.VMEM((MPS, n), jnp.float32),
            pltpu.VMEM((2 * N_DEV * G, n), jnp.float32),
            pltpu.VMEM((N_DEV, 1, 128), jnp.int32),
            pltpu.SMEM((N_DEV, 1, 128), jnp.int32),
            pltpu.SemaphoreType.DMA((N_DEV - 1,)),
            pltpu.SemaphoreType.DMA((N_DEV,)),
            pltpu.SemaphoreType.DMA((N_DEV - 1, NB)),
            pltpu.SemaphoreType.DMA((N_DEV, NB)),
            pltpu.SemaphoreType.DMA((N_DEV - 1, 2)),
            pltpu.SemaphoreType.DMA((N_DEV, 2)),
            pltpu.SemaphoreType.DMA((NB,)),
            pltpu.SemaphoreType.DMA((2,)),
            pltpu.SemaphoreType.DMA,
        ],
        compiler_params=pltpu.CompilerParams(collective_id=0),
    )(x, rank_row, ot, cnt_row)


def kernel(x, dest):
    m, n = x.shape
    i32 = jnp.int32

    onehot = (dest[:, None] == jnp.arange(N_DEV)[None, :]).astype(i32)
    cnt = jnp.sum(onehot, axis=0, dtype=i32)
    tril = jnp.tril(jnp.ones((m, m), jnp.bfloat16), -1)
    ranks = jnp.dot(tril, onehot.astype(jnp.bfloat16),
                    preferred_element_type=jnp.float32).astype(i32)
    rank_row = jnp.sum(onehot * ranks, axis=1, dtype=i32).reshape(1, m)
    ot = (jnp.arange(N_DEV, dtype=i32)[:, None] == dest[None, :]).astype(i32)

    cnt_row = jnp.zeros((1, 1, 128), i32).at[0, 0, :N_DEV].set(cnt)
    return _a2av_fused(x, rank_row, ot, cnt_row)
